# baseline (speedup 1.0000x reference)
import jax
import jax.numpy as jnp
from jax.experimental import pallas as pl
from jax.experimental.pallas import tpu as pltpu


def _fused_kernel(x_ref, w_ref, wg_ref, thr_ref, bias_ref,
                  o_ref, g_ref, coef_ref, xb_ref):
    j = pl.program_id(1)
    C = w_ref.shape[1]

    @pl.when(j == 0)
    def _gating():
        xb = x_ref[...].astype(jnp.bfloat16)
        xb_ref[...] = xb
        logits = jax.lax.dot_general(
            xb, wg_ref[...], (((1,), (1,)), ((), ())),
            preferred_element_type=jnp.float32)
        thr = jnp.transpose(thr_ref[...])
        g = (logits > thr).astype(jnp.float32)
        g_ref[...] = g
        act = float(C) * jnp.sum(g, axis=1, keepdims=True)
        denom = jnp.where(act > 0.0, act, 1.0)
        coef_ref[...] = g * (float(C) / denom)

    w = w_ref[0].astype(jnp.bfloat16)
    acc = jax.lax.dot_general(
        xb_ref[...], w, (((1,), (1,)), ((), ())),
        preferred_element_type=jnp.float32)
    E = coef_ref.shape[1]
    onehot = (jax.lax.broadcasted_iota(jnp.int32, (1, E), 1) == j)
    c = jnp.sum(jnp.where(onehot, coef_ref[...], 0.0),
                axis=1, keepdims=True)
    o_ref[...] = (acc + bias_ref[0]) * c


def kernel(x, Wg_w, Wg_b, comp_w, comp_b):
    B, D = x.shape
    E, C, _ = comp_w.shape
    NB = 4
    BM = B // NB

    U = jax.random.uniform(jax.random.key(42), (B, E), jnp.float32)
    thrT = (jnp.log(U) - jnp.log1p(-U) - Wg_b[None, :]).T

    wg = Wg_w.astype(jnp.bfloat16)
    bias3 = comp_b[:, None, :]

    out, G, _ = pl.pallas_call(
        _fused_kernel,
        grid=(NB, E),
        in_specs=[
            pl.BlockSpec((BM, D), lambda i, j: (i, 0)),
            pl.BlockSpec((1, C, D), lambda i, j: (j, 0, 0)),
            pl.BlockSpec((E, D), lambda i, j: (0, 0)),
            pl.BlockSpec((E, BM), lambda i, j: (0, i)),
            pl.BlockSpec((1, 1, C), lambda i, j: (j, 0, 0)),
        ],
        out_specs=[
            pl.BlockSpec((BM, C), lambda i, j: (i, j)),
            pl.BlockSpec((BM, E), lambda i, j: (i, 0)),
            pl.BlockSpec((BM, E), lambda i, j: (i, 0)),
        ],
        out_shape=[
            jax.ShapeDtypeStruct((B, E * C), jnp.float32),
            jax.ShapeDtypeStruct((B, E), jnp.float32),
            jax.ShapeDtypeStruct((B, E), jnp.float32),
        ],
        scratch_shapes=[pltpu.VMEM((BM, D), jnp.bfloat16)],
    )(x, comp_w, wg, thrT, bias3)
    return out, G


# speedup vs baseline: 1.1280x; 1.1280x over previous
import jax
import jax.numpy as jnp
from jax.experimental import pallas as pl
from jax.experimental.pallas import tpu as pltpu


def _fused_kernel(x_hbm, w_ref, wg_ref, thr_ref, bias_ref,
                  o_ref, g_ref, coef_ref, xf_ref, xb_ref, sem):
    i = pl.program_id(0)
    j = pl.program_id(1)
    NB = pl.num_programs(0)
    BM = xf_ref.shape[0]
    C = w_ref.shape[1]

    @pl.when(j == 0)
    def _stage_and_gate():
        @pl.when(i == 0)
        def _first():
            pltpu.make_async_copy(
                x_hbm.at[pl.ds(0, BM)], xf_ref, sem).start()
        pltpu.make_async_copy(
            x_hbm.at[pl.ds(i * BM, BM)], xf_ref, sem).wait()
        xb = xf_ref[...].astype(jnp.bfloat16)
        xb_ref[...] = xb
        logits = jax.lax.dot_general(
            xb, wg_ref[...], (((1,), (1,)), ((), ())),
            preferred_element_type=jnp.float32)
        thr = jnp.transpose(thr_ref[...])
        g = (logits > thr).astype(jnp.float32)
        g_ref[...] = g
        act = float(C) * jnp.sum(g, axis=1, keepdims=True)
        denom = jnp.where(act > 0.0, act, 1.0)
        coef_ref[...] = g * (float(C) / denom)

    @pl.when((j == 1) & (i + 1 < NB))
    def _prefetch_next():
        pltpu.make_async_copy(
            x_hbm.at[pl.ds((i + 1) * BM, BM)], xf_ref, sem).start()

    w = w_ref[0].astype(jnp.bfloat16)
    acc = jax.lax.dot_general(
        xb_ref[...], w, (((1,), (1,)), ((), ())),
        preferred_element_type=jnp.float32)
    E = coef_ref.shape[1]
    onehot = (jax.lax.broadcasted_iota(jnp.int32, (1, E), 1) == j)
    c = jnp.sum(jnp.where(onehot, coef_ref[...], 0.0),
                axis=1, keepdims=True)
    o_ref[...] = (acc + bias_ref[0]) * c


def kernel(x, Wg_w, Wg_b, comp_w, comp_b):
    B, D = x.shape
    E, C, _ = comp_w.shape
    NB = 2
    BM = B // NB

    U = jax.random.uniform(jax.random.key(42), (B, E), jnp.float32)
    thrT = (jnp.log(U) - jnp.log1p(-U) - Wg_b[None, :]).T

    wg = Wg_w.astype(jnp.bfloat16)
    bias3 = comp_b[:, None, :]

    out, G, _ = pl.pallas_call(
        _fused_kernel,
        grid=(NB, E),
        in_specs=[
            pl.BlockSpec(memory_space=pltpu.HBM),
            pl.BlockSpec((1, C, D), lambda i, j: (j, 0, 0)),
            pl.BlockSpec((E, D), lambda i, j: (0, 0)),
            pl.BlockSpec((E, BM), lambda i, j: (0, i)),
            pl.BlockSpec((1, 1, C), lambda i, j: (j, 0, 0)),
        ],
        out_specs=[
            pl.BlockSpec((BM, C), lambda i, j: (i, j)),
            pl.BlockSpec((BM, E), lambda i, j: (i, 0)),
            pl.BlockSpec((BM, E), lambda i, j: (i, 0)),
        ],
        out_shape=[
            jax.ShapeDtypeStruct((B, E * C), jnp.float32),
            jax.ShapeDtypeStruct((B, E), jnp.float32),
            jax.ShapeDtypeStruct((B, E), jnp.float32),
        ],
        scratch_shapes=[
            pltpu.VMEM((BM, D), jnp.float32),
            pltpu.VMEM((BM, D), jnp.bfloat16),
            pltpu.SemaphoreType.DMA,
        ],
    )(x, comp_w, wg, thrT, bias3)
    return out, G


# speedup vs baseline: 1.1308x; 1.0025x over previous
import jax
import jax.numpy as jnp
from jax.experimental import pallas as pl
from jax.experimental.pallas import tpu as pltpu


def _fused_kernel(x_hbm, w_ref, wg_ref, thr_ref, bias_ref,
                  o_ref, g_ref, coef_ref, xf_ref, xb_ref, sem):
    i = pl.program_id(0)
    j = pl.program_id(1)
    NB = pl.num_programs(0)
    BM = xf_ref.shape[0]
    C = w_ref.shape[1]

    @pl.when(j == 0)
    def _stage_and_gate():
        @pl.when(i == 0)
        def _first():
            pltpu.make_async_copy(
                x_hbm.at[pl.ds(0, BM)], xf_ref, sem).start()
        pltpu.make_async_copy(
            x_hbm.at[pl.ds(i * BM, BM)], xf_ref, sem).wait()
        xb = xf_ref[...].astype(jnp.bfloat16)
        xb_ref[...] = xb
        logits = jax.lax.dot_general(
            xb, wg_ref[...], (((1,), (1,)), ((), ())),
            preferred_element_type=jnp.float32)
        thr = jnp.transpose(thr_ref[...])
        g = (logits > thr).astype(jnp.float32)
        g_ref[...] = g
        act = float(C) * jnp.sum(g, axis=1, keepdims=True)
        denom = jnp.where(act > 0.0, act, 1.0)
        coef_ref[...] = g * (float(C) / denom)

    @pl.when((j == 1) & (i + 1 < NB))
    def _prefetch_next():
        pltpu.make_async_copy(
            x_hbm.at[pl.ds((i + 1) * BM, BM)], xf_ref, sem).start()

    w = w_ref[0].astype(jnp.bfloat16)
    acc = jax.lax.dot_general(
        xb_ref[...], w, (((1,), (1,)), ((), ())),
        preferred_element_type=jnp.float32)
    E = coef_ref.shape[1]
    onehot = (jax.lax.broadcasted_iota(jnp.int32, (1, E), 1) == j)
    c = jnp.sum(jnp.where(onehot, coef_ref[...], 0.0),
                axis=1, keepdims=True)
    o_ref[...] = (acc + bias_ref[0]) * c


def kernel(x, Wg_w, Wg_b, comp_w, comp_b):
    B, D = x.shape
    E, C, _ = comp_w.shape
    NB = 2
    BM = B // NB

    with jax.ensure_compile_time_eval():
        U = jax.random.uniform(jax.random.key(42), (B, E), jnp.float32)
        logitU = (jnp.log(U) - jnp.log1p(-U)).T
    thrT = logitU - Wg_b[:, None]

    wg = Wg_w.astype(jnp.bfloat16)
    bias3 = comp_b[:, None, :]

    out, G, _ = pl.pallas_call(
        _fused_kernel,
        grid=(NB, E),
        in_specs=[
            pl.BlockSpec(memory_space=pltpu.HBM),
            pl.BlockSpec((1, C, D), lambda i, j: (j, 0, 0)),
            pl.BlockSpec((E, D), lambda i, j: (0, 0)),
            pl.BlockSpec((E, BM), lambda i, j: (0, i)),
            pl.BlockSpec((1, 1, C), lambda i, j: (j, 0, 0)),
        ],
        out_specs=[
            pl.BlockSpec((BM, C), lambda i, j: (i, j)),
            pl.BlockSpec((BM, E), lambda i, j: (i, 0)),
            pl.BlockSpec((BM, E), lambda i, j: (i, 0)),
        ],
        out_shape=[
            jax.ShapeDtypeStruct((B, E * C), jnp.float32),
            jax.ShapeDtypeStruct((B, E), jnp.float32),
            jax.ShapeDtypeStruct((B, E), jnp.float32),
        ],
        scratch_shapes=[
            pltpu.VMEM((BM, D), jnp.float32),
            pltpu.VMEM((BM, D), jnp.bfloat16),
            pltpu.SemaphoreType.DMA,
        ],
    )(x, comp_w, wg, thrT, bias3)
    return out, G
